# baseline (speedup 1.0000x reference)
import functools
import jax
import jax.numpy as jnp
from jax.experimental import pallas as pl
from jax.experimental.pallas import tpu as pltpu

_BF = jnp.bfloat16
_F32 = jnp.float32


def _dot(a, b):
    return jnp.dot(a.astype(_BF), b.astype(_BF), preferred_element_type=_F32)


def _qkv_body(x_ref, g_ref, b_ref, w_ref, bias_ref, o_ref):
    x = x_ref[...]
    rms = jnp.sqrt(jnp.mean(x * x, axis=-1, keepdims=True) + 1e-6)
    xn = (x / rms) * g_ref[...] + b_ref[...]
    o_ref[...] = _dot(xn, w_ref[...]) + bias_ref[...]


def _attn_body(alpha_ref, q_ref, k_ref, v_ref, o_ref, *, q_blk, eps):
    h = pl.program_id(0)
    m = pl.program_id(1)
    q = q_ref[0]
    k = k_ref[0]
    qn = jnp.sqrt(jnp.sum(q * q, axis=-1, keepdims=True))
    kn = jnp.sqrt(jnp.sum(k * k, axis=-1, keepdims=True))
    qh = q / (qn + eps)
    kh = k / (kn + eps)
    scores = jax.lax.dot_general(
        qh.astype(_BF), kh.astype(_BF),
        (((1,), (1,)), ((), ())), preferred_element_type=_F32)
    scores = scores * alpha_ref[h]
    rows = m * q_blk + jax.lax.broadcasted_iota(jnp.int32, scores.shape, 0)
    cols = jax.lax.broadcasted_iota(jnp.int32, scores.shape, 1)
    scores = jnp.where(cols <= rows, scores, jnp.float32(-1e30))
    mx = jnp.max(scores, axis=-1, keepdims=True)
    p = jnp.exp(scores - mx)
    p = p / jnp.sum(p, axis=-1, keepdims=True)
    o_ref[0] = _dot(p, v_ref[0])


def _proj_router_body(y_ref, wp_ref, bp_ref, x_ref, wr_ref, br_ref,
                      x1_ref, w_ref):
    x1 = x_ref[...] + _dot(y_ref[...], wp_ref[...]) + bp_ref[...]
    x1_ref[...] = x1
    logits = jnp.dot(x1, wr_ref[...], preferred_element_type=_F32) + br_ref[...]
    mx = jnp.max(logits, axis=-1, keepdims=True)
    p = jnp.exp(logits - mx)
    p = p / jnp.sum(p, axis=-1, keepdims=True)
    w_ref[...] = p / jnp.sum(p, axis=-1, keepdims=True)


def _h_body(x_ref, w1_ref, b1_ref, h_ref):
    h_ref[0] = (_dot(x_ref[...], w1_ref[0]) + b1_ref[0]).astype(_BF)


def _swiglu_body(h_ref, waa_ref, baa_ref, wab_ref, bab_ref, t_ref,
                 wcat_ref, *, c_half):
    wcat_ref[:, :c_half] = waa_ref[0].astype(_BF)
    wcat_ref[:, c_half:] = wab_ref[0].astype(_BF)
    comb = jnp.dot(h_ref[0], wcat_ref[...], preferred_element_type=_F32)
    a = comb[:, :c_half] + baa_ref[0]
    bc = comb[:, c_half:] + bab_ref[0]
    t_ref[0] = ((bc * jax.nn.sigmoid(bc)) * a).astype(_BF)


def _h2_body(t_ref, wb_ref, bb_ref, h2_ref):
    h2_ref[0] = (_dot(t_ref[0], wb_ref[0]) + bb_ref[0]).astype(_BF)


def _final_body(h2_ref, w2_ref, b2_ref, x_ref, wts_ref, o_ref):
    e = pl.program_id(0)
    contrib = (_dot(h2_ref[0], w2_ref[0]) + b2_ref[0]) * wts_ref[0]

    @pl.when(e == 0)
    def _():
        o_ref[...] = x_ref[...] + contrib

    @pl.when(e != 0)
    def _():
        o_ref[...] += contrib


def kernel(x, g, b, Wqkv, bqkv, alpha, Wproj, bproj, Wr, br,
           W1, b1, Wa, ba, Wb, bb, W2, b2):
    B, T, D = x.shape
    NH = alpha.shape[0]
    HD = D // NH
    E = Wr.shape[1]
    DFF = W1.shape[2]

    x2 = x.reshape(T, D)
    g2 = g.reshape(1, D)
    b2d = b.reshape(1, D)
    bqkv2 = bqkv.reshape(1, 3 * D)
    bproj2 = bproj.reshape(1, D)
    br2 = br.reshape(1, E)

    row_blk = 256 if T % 256 == 0 else T
    n_rows = T // row_blk

    qkv = pl.pallas_call(
        _qkv_body,
        grid=(n_rows,),
        in_specs=[
            pl.BlockSpec((row_blk, D), lambda i: (i, 0)),
            pl.BlockSpec((1, D), lambda i: (0, 0)),
            pl.BlockSpec((1, D), lambda i: (0, 0)),
            pl.BlockSpec((D, 3 * D), lambda i: (0, 0)),
            pl.BlockSpec((1, 3 * D), lambda i: (0, 0)),
        ],
        out_specs=pl.BlockSpec((row_blk, 3 * D), lambda i: (i, 0)),
        out_shape=jax.ShapeDtypeStruct((T, 3 * D), _F32),
    )(x2, g2, b2d, Wqkv, bqkv2)

    qh_, kh_, vh_ = [
        jnp.transpose(qkv[:, i * D:(i + 1) * D].reshape(T, NH, HD), (1, 0, 2))
        for i in range(3)
    ]
    q_blk = row_blk
    yh = pl.pallas_call(
        functools.partial(_attn_body, q_blk=q_blk, eps=1e-5),
        grid=(NH, n_rows),
        in_specs=[
            pl.BlockSpec(memory_space=pltpu.SMEM),
            pl.BlockSpec((1, q_blk, HD), lambda h, m: (h, m, 0)),
            pl.BlockSpec((1, T, HD), lambda h, m: (h, 0, 0)),
            pl.BlockSpec((1, T, HD), lambda h, m: (h, 0, 0)),
        ],
        out_specs=pl.BlockSpec((1, q_blk, HD), lambda h, m: (h, m, 0)),
        out_shape=jax.ShapeDtypeStruct((NH, T, HD), _F32),
        compiler_params=pltpu.CompilerParams(
            dimension_semantics=("arbitrary", "arbitrary")),
    )(alpha, qh_, kh_, vh_)
    y = jnp.transpose(yh, (1, 0, 2)).reshape(T, D)

    x1, w = pl.pallas_call(
        _proj_router_body,
        grid=(n_rows,),
        in_specs=[
            pl.BlockSpec((row_blk, D), lambda i: (i, 0)),
            pl.BlockSpec((D, D), lambda i: (0, 0)),
            pl.BlockSpec((1, D), lambda i: (0, 0)),
            pl.BlockSpec((row_blk, D), lambda i: (i, 0)),
            pl.BlockSpec((D, E), lambda i: (0, 0)),
            pl.BlockSpec((1, E), lambda i: (0, 0)),
        ],
        out_specs=[
            pl.BlockSpec((row_blk, D), lambda i: (i, 0)),
            pl.BlockSpec((row_blk, E), lambda i: (i, 0)),
        ],
        out_shape=[
            jax.ShapeDtypeStruct((T, D), _F32),
            jax.ShapeDtypeStruct((T, E), _F32),
        ],
    )(y, Wproj, bproj2, x2, Wr, br2)

    wts = jnp.transpose(w)[:, :, None]

    n_h = 2
    CH = DFF // n_h
    h_all = pl.pallas_call(
        _h_body,
        grid=(E, n_h),
        in_specs=[
            pl.BlockSpec((T, D), lambda e, j: (0, 0)),
            pl.BlockSpec((1, D, CH), lambda e, j: (e, 0, j)),
            pl.BlockSpec((1, 1, CH), lambda e, j: (e, 0, j)),
        ],
        out_specs=pl.BlockSpec((1, T, CH), lambda e, j: (e, 0, j)),
        out_shape=jax.ShapeDtypeStruct((E, T, DFF), _BF),
        compiler_params=pltpu.CompilerParams(
            dimension_semantics=("arbitrary", "arbitrary")),
    )(x1, W1, b1[:, None, :])

    n_c = 6
    C = DFF // n_c
    t_all = pl.pallas_call(
        functools.partial(_swiglu_body, c_half=C),
        grid=(E, n_c),
        in_specs=[
            pl.BlockSpec((1, T, DFF), lambda e, j: (e, 0, 0),
                         pipeline_mode=pl.Buffered(buffer_count=1)),
            pl.BlockSpec((1, DFF, C), lambda e, j: (e, 0, j)),
            pl.BlockSpec((1, 1, C), lambda e, j: (e, 0, j)),
            pl.BlockSpec((1, DFF, C), lambda e, j: (e, 0, n_c + j)),
            pl.BlockSpec((1, 1, C), lambda e, j: (e, 0, n_c + j)),
        ],
        out_specs=pl.BlockSpec((1, T, C), lambda e, j: (e, 0, j)),
        out_shape=jax.ShapeDtypeStruct((E, T, DFF), _BF),
        scratch_shapes=[
            pltpu.VMEM((DFF, 2 * C), _BF),
        ],
        compiler_params=pltpu.CompilerParams(
            dimension_semantics=("arbitrary", "arbitrary")),
    )(h_all, Wa, ba[:, None, :], Wa, ba[:, None, :])

    n_c2 = 4
    C2 = DFF // n_c2
    h2_all = pl.pallas_call(
        _h2_body,
        grid=(E, n_c2),
        in_specs=[
            pl.BlockSpec((1, T, DFF), lambda e, c: (e, 0, 0),
                         pipeline_mode=pl.Buffered(buffer_count=1)),
            pl.BlockSpec((1, DFF, C2), lambda e, c: (e, 0, c)),
            pl.BlockSpec((1, 1, C2), lambda e, c: (e, 0, c)),
        ],
        out_specs=pl.BlockSpec((1, T, C2), lambda e, c: (e, 0, c)),
        out_shape=jax.ShapeDtypeStruct((E, T, DFF), _BF),
        compiler_params=pltpu.CompilerParams(
            dimension_semantics=("arbitrary", "arbitrary")),
    )(t_all, Wb, bb[:, None, :])

    out = pl.pallas_call(
        _final_body,
        grid=(E,),
        in_specs=[
            pl.BlockSpec((1, T, DFF), lambda e: (e, 0, 0),
                         pipeline_mode=pl.Buffered(buffer_count=1)),
            pl.BlockSpec((1, DFF, D), lambda e: (e, 0, 0),
                         pipeline_mode=pl.Buffered(buffer_count=1)),
            pl.BlockSpec((1, 1, D), lambda e: (e, 0, 0)),
            pl.BlockSpec((T, D), lambda e: (0, 0),
                         pipeline_mode=pl.Buffered(buffer_count=1)),
            pl.BlockSpec((1, T, 1), lambda e: (e, 0, 0)),
        ],
        out_specs=pl.BlockSpec((T, D), lambda e: (0, 0)),
        out_shape=jax.ShapeDtypeStruct((T, D), _F32),
        compiler_params=pltpu.CompilerParams(
            dimension_semantics=("arbitrary",)),
    )(h2_all, W2, b2[:, None, :], x1, wts)

    return out.reshape(B, T, D)


# speedup vs baseline: 1.4985x; 1.4985x over previous
import functools
import jax
import jax.numpy as jnp
from jax.experimental import pallas as pl
from jax.experimental.pallas import tpu as pltpu

_BF = jnp.bfloat16
_F32 = jnp.float32


def _dot(a, b):
    return jnp.dot(a.astype(_BF), b.astype(_BF), preferred_element_type=_F32)


def _qkv_body(x_ref, g_ref, b_ref, w_ref, bias_ref, o_ref):
    x = x_ref[...]
    rms = jnp.sqrt(jnp.mean(x * x, axis=-1, keepdims=True) + 1e-6)
    xn = (x / rms) * g_ref[...] + b_ref[...]
    o_ref[...] = _dot(xn, w_ref[...]) + bias_ref[...]


def _attn_body(alpha_ref, q_ref, k_ref, v_ref, o_ref, *, q_blk, eps):
    h = pl.program_id(0)
    m = pl.program_id(1)
    q = q_ref[0]
    k = k_ref[0]
    qn = jnp.sqrt(jnp.sum(q * q, axis=-1, keepdims=True))
    kn = jnp.sqrt(jnp.sum(k * k, axis=-1, keepdims=True))
    qh = q / (qn + eps)
    kh = k / (kn + eps)
    scores = jax.lax.dot_general(
        qh.astype(_BF), kh.astype(_BF),
        (((1,), (1,)), ((), ())), preferred_element_type=_F32)
    scores = scores * alpha_ref[h]
    rows = m * q_blk + jax.lax.broadcasted_iota(jnp.int32, scores.shape, 0)
    cols = jax.lax.broadcasted_iota(jnp.int32, scores.shape, 1)
    scores = jnp.where(cols <= rows, scores, jnp.float32(-1e30))
    mx = jnp.max(scores, axis=-1, keepdims=True)
    p = jnp.exp(scores - mx)
    p = p / jnp.sum(p, axis=-1, keepdims=True)
    o_ref[0] = _dot(p, v_ref[0])


def _proj_router_body(y_ref, wp_ref, bp_ref, x_ref, wr_ref, br_ref,
                      x1_ref, x1b_ref, w_ref):
    x1 = x_ref[...] + _dot(y_ref[...], wp_ref[...]) + bp_ref[...]
    x1_ref[...] = x1
    x1b_ref[...] = x1.astype(_BF)
    logits = jnp.dot(x1, wr_ref[...], preferred_element_type=_F32) + br_ref[...]
    mx = jnp.max(logits, axis=-1, keepdims=True)
    p = jnp.exp(logits - mx)
    p = p / jnp.sum(p, axis=-1, keepdims=True)
    w_ref[...] = p / jnp.sum(p, axis=-1, keepdims=True)


def _wfa_body(w1_ref, wa_ref, b1_ref, ba_ref, wfa_ref, bfa_ref):
    wab = wa_ref[0].astype(_BF)
    wfa_ref[0] = jnp.dot(w1_ref[0].astype(_BF), wab,
                         preferred_element_type=_F32).astype(_BF)
    bfa_ref[0] = jnp.dot(b1_ref[0].astype(_BF), wab,
                         preferred_element_type=_F32) + ba_ref[0]


def _wb2_body(wb_ref, w2_ref, bb_ref, b2_ref, wb2_ref, bv2_ref):
    r = pl.program_id(1)
    w2b = w2_ref[0].astype(_BF)
    wb2_ref[0] = jnp.dot(wb_ref[0].astype(_BF), w2b,
                         preferred_element_type=_F32).astype(_BF)

    @pl.when(r == 0)
    def _():
        bv2_ref[0] = jnp.dot(bb_ref[0].astype(_BF), w2b,
                             preferred_element_type=_F32) + b2_ref[0]


def _apply_body(xb_ref, wfaa_ref, bfaa_ref, wfab_ref, bfab_ref,
                wb2_ref, bv2_ref, x_ref, wts_ref, o_ref, acc_ref,
                *, n_chunks):
    e = pl.program_id(0)
    j = pl.program_id(1)

    @pl.when(j == 0)
    def _():
        acc_ref[...] = jnp.broadcast_to(bv2_ref[0], acc_ref.shape)

    xb = xb_ref[...]
    a = jnp.dot(xb, wfaa_ref[0], preferred_element_type=_F32) + bfaa_ref[0]
    bc = jnp.dot(xb, wfab_ref[0], preferred_element_type=_F32) + bfab_ref[0]
    t = ((bc * jax.nn.sigmoid(bc)) * a).astype(_BF)
    acc_ref[...] += jnp.dot(t, wb2_ref[0], preferred_element_type=_F32)

    @pl.when(j == n_chunks - 1)
    def _():
        contrib = acc_ref[...] * wts_ref[0]

        @pl.when(e == 0)
        def _():
            o_ref[...] = x_ref[...] + contrib

        @pl.when(e != 0)
        def _():
            o_ref[...] += contrib


def kernel(x, g, b, Wqkv, bqkv, alpha, Wproj, bproj, Wr, br,
           W1, b1, Wa, ba, Wb, bb, W2, b2):
    B, T, D = x.shape
    NH = alpha.shape[0]
    HD = D // NH
    E = Wr.shape[1]
    DFF = W1.shape[2]

    x2 = x.reshape(T, D)
    g2 = g.reshape(1, D)
    b2d = b.reshape(1, D)
    bqkv2 = bqkv.reshape(1, 3 * D)
    bproj2 = bproj.reshape(1, D)
    br2 = br.reshape(1, E)

    row_blk = 256 if T % 256 == 0 else T
    n_rows = T // row_blk

    qkv = pl.pallas_call(
        _qkv_body,
        grid=(n_rows,),
        in_specs=[
            pl.BlockSpec((row_blk, D), lambda i: (i, 0)),
            pl.BlockSpec((1, D), lambda i: (0, 0)),
            pl.BlockSpec((1, D), lambda i: (0, 0)),
            pl.BlockSpec((D, 3 * D), lambda i: (0, 0)),
            pl.BlockSpec((1, 3 * D), lambda i: (0, 0)),
        ],
        out_specs=pl.BlockSpec((row_blk, 3 * D), lambda i: (i, 0)),
        out_shape=jax.ShapeDtypeStruct((T, 3 * D), _F32),
    )(x2, g2, b2d, Wqkv, bqkv2)

    qh_, kh_, vh_ = [
        jnp.transpose(qkv[:, i * D:(i + 1) * D].reshape(T, NH, HD), (1, 0, 2))
        for i in range(3)
    ]
    q_blk = row_blk
    yh = pl.pallas_call(
        functools.partial(_attn_body, q_blk=q_blk, eps=1e-5),
        grid=(NH, n_rows),
        in_specs=[
            pl.BlockSpec(memory_space=pltpu.SMEM),
            pl.BlockSpec((1, q_blk, HD), lambda h, m: (h, m, 0)),
            pl.BlockSpec((1, T, HD), lambda h, m: (h, 0, 0)),
            pl.BlockSpec((1, T, HD), lambda h, m: (h, 0, 0)),
        ],
        out_specs=pl.BlockSpec((1, q_blk, HD), lambda h, m: (h, m, 0)),
        out_shape=jax.ShapeDtypeStruct((NH, T, HD), _F32),
        compiler_params=pltpu.CompilerParams(
            dimension_semantics=("arbitrary", "arbitrary")),
    )(alpha, qh_, kh_, vh_)
    y = jnp.transpose(yh, (1, 0, 2)).reshape(T, D)

    x1, x1b, w = pl.pallas_call(
        _proj_router_body,
        grid=(n_rows,),
        in_specs=[
            pl.BlockSpec((row_blk, D), lambda i: (i, 0)),
            pl.BlockSpec((D, D), lambda i: (0, 0)),
            pl.BlockSpec((1, D), lambda i: (0, 0)),
            pl.BlockSpec((row_blk, D), lambda i: (i, 0)),
            pl.BlockSpec((D, E), lambda i: (0, 0)),
            pl.BlockSpec((1, E), lambda i: (0, 0)),
        ],
        out_specs=[
            pl.BlockSpec((row_blk, D), lambda i: (i, 0)),
            pl.BlockSpec((row_blk, D), lambda i: (i, 0)),
            pl.BlockSpec((row_blk, E), lambda i: (i, 0)),
        ],
        out_shape=[
            jax.ShapeDtypeStruct((T, D), _F32),
            jax.ShapeDtypeStruct((T, D), _BF),
            jax.ShapeDtypeStruct((T, E), _F32),
        ],
    )(y, Wproj, bproj2, x2, Wr, br2)

    wts = jnp.transpose(w)[:, :, None]

    n1 = 6
    C1 = (2 * DFF) // n1
    Wfa, bfa = pl.pallas_call(
        _wfa_body,
        grid=(E, n1),
        in_specs=[
            pl.BlockSpec((1, D, DFF), lambda e, j: (e, 0, 0),
                         pipeline_mode=pl.Buffered(buffer_count=1)),
            pl.BlockSpec((1, DFF, C1), lambda e, j: (e, 0, j)),
            pl.BlockSpec((1, 1, DFF), lambda e, j: (e, 0, 0)),
            pl.BlockSpec((1, 1, C1), lambda e, j: (e, 0, j)),
        ],
        out_specs=[
            pl.BlockSpec((1, D, C1), lambda e, j: (e, 0, j)),
            pl.BlockSpec((1, 1, C1), lambda e, j: (e, 0, j)),
        ],
        out_shape=[
            jax.ShapeDtypeStruct((E, D, 2 * DFF), _BF),
            jax.ShapeDtypeStruct((E, 1, 2 * DFF), _F32),
        ],
        compiler_params=pltpu.CompilerParams(
            dimension_semantics=("arbitrary", "arbitrary")),
    )(W1, Wa, b1[:, None, :], ba[:, None, :])

    n2 = 4
    R2 = DFF // n2
    Wb2, bv2 = pl.pallas_call(
        _wb2_body,
        grid=(E, n2),
        in_specs=[
            pl.BlockSpec((1, R2, DFF), lambda e, r: (e, r, 0)),
            pl.BlockSpec((1, DFF, D), lambda e, r: (e, 0, 0),
                         pipeline_mode=pl.Buffered(buffer_count=1)),
            pl.BlockSpec((1, 1, DFF), lambda e, r: (e, 0, 0)),
            pl.BlockSpec((1, 1, D), lambda e, r: (e, 0, 0)),
        ],
        out_specs=[
            pl.BlockSpec((1, R2, D), lambda e, r: (e, r, 0)),
            pl.BlockSpec((1, 1, D), lambda e, r: (e, 0, 0)),
        ],
        out_shape=[
            jax.ShapeDtypeStruct((E, DFF, D), _BF),
            jax.ShapeDtypeStruct((E, 1, D), _F32),
        ],
        compiler_params=pltpu.CompilerParams(
            dimension_semantics=("arbitrary", "arbitrary")),
    )(Wb, W2, bb[:, None, :], b2[:, None, :])

    n_c = 4
    C = DFF // n_c
    out = pl.pallas_call(
        functools.partial(_apply_body, n_chunks=n_c),
        grid=(E, n_c),
        in_specs=[
            pl.BlockSpec((T, D), lambda e, j: (0, 0),
                         pipeline_mode=pl.Buffered(buffer_count=1)),
            pl.BlockSpec((1, D, C), lambda e, j: (e, 0, j)),
            pl.BlockSpec((1, 1, C), lambda e, j: (e, 0, j)),
            pl.BlockSpec((1, D, C), lambda e, j: (e, 0, n_c + j)),
            pl.BlockSpec((1, 1, C), lambda e, j: (e, 0, n_c + j)),
            pl.BlockSpec((1, C, D), lambda e, j: (e, j, 0)),
            pl.BlockSpec((1, 1, D), lambda e, j: (e, 0, 0)),
            pl.BlockSpec((T, D), lambda e, j: (0, 0),
                         pipeline_mode=pl.Buffered(buffer_count=1)),
            pl.BlockSpec((1, T, 1), lambda e, j: (e, 0, 0)),
        ],
        out_specs=pl.BlockSpec((T, D), lambda e, j: (0, 0)),
        out_shape=jax.ShapeDtypeStruct((T, D), _F32),
        scratch_shapes=[
            pltpu.VMEM((T, D), _F32),
        ],
        compiler_params=pltpu.CompilerParams(
            dimension_semantics=("arbitrary", "arbitrary")),
    )(x1b, Wfa, bfa, Wfa, bfa, Wb2, bv2, x1, wts)

    return out.reshape(B, T, D)


# speedup vs baseline: 1.6874x; 1.1260x over previous
import functools
import jax
import jax.numpy as jnp
from jax.experimental import pallas as pl
from jax.experimental.pallas import tpu as pltpu

_BF = jnp.bfloat16
_F32 = jnp.float32


def _dot(a, b):
    return jnp.dot(a.astype(_BF), b.astype(_BF), preferred_element_type=_F32)


def _qkv_body(x_ref, g_ref, b_ref, w_ref, bias_ref, o_ref):
    x = x_ref[...]
    rms = jnp.sqrt(jnp.mean(x * x, axis=-1, keepdims=True) + 1e-6)
    xn = (x / rms) * g_ref[...] + b_ref[...]
    o_ref[...] = _dot(xn, w_ref[...]) + bias_ref[...]


def _attn_body(alpha_ref, q_ref, k_ref, v_ref, o_ref, *, q_blk, hd, n_head,
               eps):
    m = pl.program_id(0)
    rows = None
    for h in range(n_head):
        q = q_ref[:, h * hd:(h + 1) * hd]
        k = k_ref[:, h * hd:(h + 1) * hd]
        qn = jnp.sqrt(jnp.sum(q * q, axis=-1, keepdims=True))
        kn = jnp.sqrt(jnp.sum(k * k, axis=-1, keepdims=True))
        qh = q / (qn + eps)
        kh = k / (kn + eps)
        scores = jax.lax.dot_general(
            qh.astype(_BF), kh.astype(_BF),
            (((1,), (1,)), ((), ())), preferred_element_type=_F32)
        scores = scores * alpha_ref[h]
        if rows is None:
            rows = m * q_blk + jax.lax.broadcasted_iota(
                jnp.int32, scores.shape, 0)
            cols = jax.lax.broadcasted_iota(jnp.int32, scores.shape, 1)
            causal = cols <= rows
        scores = jnp.where(causal, scores, jnp.float32(-1e30))
        mx = jnp.max(scores, axis=-1, keepdims=True)
        p = jnp.exp(scores - mx)
        p = p / jnp.sum(p, axis=-1, keepdims=True)
        o_ref[:, h * hd:(h + 1) * hd] = _dot(p, v_ref[:, h * hd:(h + 1) * hd])


def _proj_router_body(y_ref, wp_ref, bp_ref, x_ref, wr_ref, br_ref,
                      x1_ref, x1b_ref, w_ref):
    x1 = x_ref[...] + _dot(y_ref[...], wp_ref[...]) + bp_ref[...]
    x1_ref[...] = x1
    x1b_ref[...] = x1.astype(_BF)
    logits = jnp.dot(x1, wr_ref[...], preferred_element_type=_F32) + br_ref[...]
    mx = jnp.max(logits, axis=-1, keepdims=True)
    p = jnp.exp(logits - mx)
    p = p / jnp.sum(p, axis=-1, keepdims=True)
    p = p / jnp.sum(p, axis=-1, keepdims=True)
    w_ref[...] = jnp.transpose(p, (1, 0))[:, :, None]


def _wfa_body(w1_ref, wa_ref, b1_ref, ba_ref, wfa_ref, bfa_ref):
    wab = wa_ref[0].astype(_BF)
    wfa_ref[0] = jnp.dot(w1_ref[0].astype(_BF), wab,
                         preferred_element_type=_F32).astype(_BF)
    bfa_ref[0] = jnp.dot(b1_ref[0].astype(_BF), wab,
                         preferred_element_type=_F32) + ba_ref[0]


def _wb2_body(wb_ref, w2_ref, bb_ref, b2_ref, wb2_ref, bv2_ref):
    r = pl.program_id(1)
    w2b = w2_ref[0].astype(_BF)
    wb2_ref[0] = jnp.dot(wb_ref[0].astype(_BF), w2b,
                         preferred_element_type=_F32).astype(_BF)

    @pl.when(r == 0)
    def _():
        bv2_ref[0] = jnp.dot(bb_ref[0].astype(_BF), w2b,
                             preferred_element_type=_F32) + b2_ref[0]


def _apply_body(xb_ref, wfaa_ref, bfaa_ref, wfab_ref, bfab_ref,
                wb2_ref, bv2_ref, x_ref, wts_ref, o_ref, acc_ref,
                *, n_chunks):
    e = pl.program_id(0)
    j = pl.program_id(1)

    @pl.when(j == 0)
    def _():
        acc_ref[...] = jnp.broadcast_to(bv2_ref[0], acc_ref.shape)

    xb = xb_ref[...]
    a = jnp.dot(xb, wfaa_ref[0], preferred_element_type=_F32) + bfaa_ref[0]
    bc = jnp.dot(xb, wfab_ref[0], preferred_element_type=_F32) + bfab_ref[0]
    t = ((bc * jax.nn.sigmoid(bc)) * a).astype(_BF)
    acc_ref[...] += jnp.dot(t, wb2_ref[0], preferred_element_type=_F32)

    @pl.when(j == n_chunks - 1)
    def _():
        contrib = acc_ref[...] * wts_ref[0]

        @pl.when(e == 0)
        def _():
            o_ref[...] = x_ref[...] + contrib

        @pl.when(e != 0)
        def _():
            o_ref[...] += contrib


def kernel(x, g, b, Wqkv, bqkv, alpha, Wproj, bproj, Wr, br,
           W1, b1, Wa, ba, Wb, bb, W2, b2):
    B, T, D = x.shape
    NH = alpha.shape[0]
    HD = D // NH
    E = Wr.shape[1]
    DFF = W1.shape[2]

    x2 = x.reshape(T, D)
    g2 = g.reshape(1, D)
    b2d = b.reshape(1, D)
    bqkv2 = bqkv.reshape(1, 3 * D)
    bproj2 = bproj.reshape(1, D)
    br2 = br.reshape(1, E)

    row_blk = 256 if T % 256 == 0 else T
    n_rows = T // row_blk

    qkv = pl.pallas_call(
        _qkv_body,
        grid=(n_rows,),
        in_specs=[
            pl.BlockSpec((row_blk, D), lambda i: (i, 0)),
            pl.BlockSpec((1, D), lambda i: (0, 0)),
            pl.BlockSpec((1, D), lambda i: (0, 0)),
            pl.BlockSpec((D, 3 * D), lambda i: (0, 0)),
            pl.BlockSpec((1, 3 * D), lambda i: (0, 0)),
        ],
        out_specs=pl.BlockSpec((row_blk, 3 * D), lambda i: (i, 0)),
        out_shape=jax.ShapeDtypeStruct((T, 3 * D), _F32),
    )(x2, g2, b2d, Wqkv, bqkv2)

    q_blk = row_blk
    y = pl.pallas_call(
        functools.partial(_attn_body, q_blk=q_blk, hd=HD, n_head=NH,
                          eps=1e-5),
        grid=(n_rows,),
        in_specs=[
            pl.BlockSpec(memory_space=pltpu.SMEM),
            pl.BlockSpec((q_blk, D), lambda m: (m, 0)),
            pl.BlockSpec((T, D), lambda m: (0, 1),
                         pipeline_mode=pl.Buffered(buffer_count=1)),
            pl.BlockSpec((T, D), lambda m: (0, 2),
                         pipeline_mode=pl.Buffered(buffer_count=1)),
        ],
        out_specs=pl.BlockSpec((q_blk, D), lambda m: (m, 0)),
        out_shape=jax.ShapeDtypeStruct((T, D), _F32),
        compiler_params=pltpu.CompilerParams(
            dimension_semantics=("arbitrary",)),
    )(alpha, qkv, qkv, qkv)

    x1, x1b, w = pl.pallas_call(
        _proj_router_body,
        grid=(n_rows,),
        in_specs=[
            pl.BlockSpec((row_blk, D), lambda i: (i, 0)),
            pl.BlockSpec((D, D), lambda i: (0, 0)),
            pl.BlockSpec((1, D), lambda i: (0, 0)),
            pl.BlockSpec((row_blk, D), lambda i: (i, 0)),
            pl.BlockSpec((D, E), lambda i: (0, 0)),
            pl.BlockSpec((1, E), lambda i: (0, 0)),
        ],
        out_specs=[
            pl.BlockSpec((row_blk, D), lambda i: (i, 0)),
            pl.BlockSpec((row_blk, D), lambda i: (i, 0)),
            pl.BlockSpec((E, row_blk, 1), lambda i: (0, i, 0)),
        ],
        out_shape=[
            jax.ShapeDtypeStruct((T, D), _F32),
            jax.ShapeDtypeStruct((T, D), _BF),
            jax.ShapeDtypeStruct((E, T, 1), _F32),
        ],
    )(y, Wproj, bproj2, x2, Wr, br2)
    wts = w

    n1 = 6
    C1 = (2 * DFF) // n1
    Wfa, bfa = pl.pallas_call(
        _wfa_body,
        grid=(E, n1),
        in_specs=[
            pl.BlockSpec((1, D, DFF), lambda e, j: (e, 0, 0)),
            pl.BlockSpec((1, DFF, C1), lambda e, j: (e, 0, j)),
            pl.BlockSpec((1, 1, DFF), lambda e, j: (e, 0, 0)),
            pl.BlockSpec((1, 1, C1), lambda e, j: (e, 0, j)),
        ],
        out_specs=[
            pl.BlockSpec((1, D, C1), lambda e, j: (e, 0, j)),
            pl.BlockSpec((1, 1, C1), lambda e, j: (e, 0, j)),
        ],
        out_shape=[
            jax.ShapeDtypeStruct((E, D, 2 * DFF), _BF),
            jax.ShapeDtypeStruct((E, 1, 2 * DFF), _F32),
        ],
        compiler_params=pltpu.CompilerParams(
            dimension_semantics=("arbitrary", "arbitrary")),
    )(W1, Wa, b1[:, None, :], ba[:, None, :])

    n2 = 4
    R2 = DFF // n2
    Wb2, bv2 = pl.pallas_call(
        _wb2_body,
        grid=(E, n2),
        in_specs=[
            pl.BlockSpec((1, R2, DFF), lambda e, r: (e, r, 0)),
            pl.BlockSpec((1, DFF, D), lambda e, r: (e, 0, 0)),
            pl.BlockSpec((1, 1, DFF), lambda e, r: (e, 0, 0)),
            pl.BlockSpec((1, 1, D), lambda e, r: (e, 0, 0)),
        ],
        out_specs=[
            pl.BlockSpec((1, R2, D), lambda e, r: (e, r, 0)),
            pl.BlockSpec((1, 1, D), lambda e, r: (e, 0, 0)),
        ],
        out_shape=[
            jax.ShapeDtypeStruct((E, DFF, D), _BF),
            jax.ShapeDtypeStruct((E, 1, D), _F32),
        ],
        compiler_params=pltpu.CompilerParams(
            dimension_semantics=("arbitrary", "arbitrary")),
    )(Wb, W2, bb[:, None, :], b2[:, None, :])

    n_c = 3
    C = DFF // n_c
    out = pl.pallas_call(
        functools.partial(_apply_body, n_chunks=n_c),
        grid=(E, n_c),
        in_specs=[
            pl.BlockSpec((T, D), lambda e, j: (0, 0),
                         pipeline_mode=pl.Buffered(buffer_count=1)),
            pl.BlockSpec((1, D, C), lambda e, j: (e, 0, j)),
            pl.BlockSpec((1, 1, C), lambda e, j: (e, 0, j)),
            pl.BlockSpec((1, D, C), lambda e, j: (e, 0, n_c + j)),
            pl.BlockSpec((1, 1, C), lambda e, j: (e, 0, n_c + j)),
            pl.BlockSpec((1, C, D), lambda e, j: (e, j, 0)),
            pl.BlockSpec((1, 1, D), lambda e, j: (e, 0, 0)),
            pl.BlockSpec((T, D), lambda e, j: (0, 0),
                         pipeline_mode=pl.Buffered(buffer_count=1)),
            pl.BlockSpec((1, T, 1), lambda e, j: (e, 0, 0)),
        ],
        out_specs=pl.BlockSpec((T, D), lambda e, j: (0, 0)),
        out_shape=jax.ShapeDtypeStruct((T, D), _F32),
        scratch_shapes=[
            pltpu.VMEM((T, D), _F32),
        ],
        compiler_params=pltpu.CompilerParams(
            dimension_semantics=("arbitrary", "arbitrary")),
    )(x1b, Wfa, bfa, Wfa, bfa, Wb2, bv2, x1, wts)

    return out.reshape(B, T, D)


# speedup vs baseline: 1.7186x; 1.0185x over previous
import functools
import jax
import jax.numpy as jnp
from jax.experimental import pallas as pl
from jax.experimental.pallas import tpu as pltpu

_BF = jnp.bfloat16
_F32 = jnp.float32


def _dot(a, b):
    return jnp.dot(a.astype(_BF), b.astype(_BF), preferred_element_type=_F32)


def _qkv_body(x_ref, g_ref, b_ref, w_ref, bias_ref, o_ref):
    x = x_ref[...]
    rms = jnp.sqrt(jnp.mean(x * x, axis=-1, keepdims=True) + 1e-6)
    xn = (x / rms) * g_ref[...] + b_ref[...]
    o_ref[...] = _dot(xn, w_ref[...]) + bias_ref[...]


def _attn_body(alpha_ref, q_ref, k_ref, v_ref, o_ref, *, q_blk, hd, n_head,
               eps, m_off):
    m = pl.program_id(0) + m_off
    rows = None
    for h in range(n_head):
        q = q_ref[:, h * hd:(h + 1) * hd]
        k = k_ref[:, h * hd:(h + 1) * hd]
        qn = jnp.sqrt(jnp.sum(q * q, axis=-1, keepdims=True))
        kn = jnp.sqrt(jnp.sum(k * k, axis=-1, keepdims=True))
        qh = q / (qn + eps)
        kh = k / (kn + eps)
        scores = jax.lax.dot_general(
            qh.astype(_BF), kh.astype(_BF),
            (((1,), (1,)), ((), ())), preferred_element_type=_F32)
        scores = scores * alpha_ref[h]
        if rows is None:
            rows = m * q_blk + jax.lax.broadcasted_iota(
                jnp.int32, scores.shape, 0)
            cols = jax.lax.broadcasted_iota(jnp.int32, scores.shape, 1)
            causal = cols <= rows
        scores = jnp.where(causal, scores, jnp.float32(-1e30))
        mx = jnp.max(scores, axis=-1, keepdims=True)
        p = jnp.exp(scores - mx)
        p = p / jnp.sum(p, axis=-1, keepdims=True)
        o_ref[:, h * hd:(h + 1) * hd] = _dot(p, v_ref[:, h * hd:(h + 1) * hd])


def _proj_router_body(y_ref, wp_ref, bp_ref, x_ref, wr_ref, br_ref,
                      x1_ref, x1b_ref, w_ref):
    x1 = x_ref[...] + _dot(y_ref[...], wp_ref[...]) + bp_ref[...]
    x1_ref[...] = x1
    x1b_ref[...] = x1.astype(_BF)
    logits = jnp.dot(x1, wr_ref[...], preferred_element_type=_F32) + br_ref[...]
    mx = jnp.max(logits, axis=-1, keepdims=True)
    p = jnp.exp(logits - mx)
    p = p / jnp.sum(p, axis=-1, keepdims=True)
    p = p / jnp.sum(p, axis=-1, keepdims=True)
    w_ref[...] = jnp.transpose(p, (1, 0))[:, :, None]


def _wfa_body(w1_ref, wa_ref, b1_ref, ba_ref, wfa_ref, bfa_ref):
    wab = wa_ref[0].astype(_BF)
    wfa_ref[0] = jnp.dot(w1_ref[0].astype(_BF), wab,
                         preferred_element_type=_F32).astype(_BF)
    bfa_ref[0] = jnp.dot(b1_ref[0].astype(_BF), wab,
                         preferred_element_type=_F32) + ba_ref[0]


def _wb2_body(wb_ref, w2_ref, bb_ref, b2_ref, wb2_ref, bv2_ref):
    r = pl.program_id(1)
    w2b = w2_ref[0].astype(_BF)
    wb2_ref[0] = jnp.dot(wb_ref[0].astype(_BF), w2b,
                         preferred_element_type=_F32).astype(_BF)

    @pl.when(r == 0)
    def _():
        bv2_ref[0] = jnp.dot(bb_ref[0].astype(_BF), w2b,
                             preferred_element_type=_F32) + b2_ref[0]


def _apply_body(xb_ref, wfa_ref, bfa_ref, wb2_ref, bv2_ref, x_ref, wts_ref,
                o_ref, acc_ref, *, c_half, n_chunks):
    e = pl.program_id(0)
    j = pl.program_id(1)

    @pl.when(j == 0)
    def _():
        acc_ref[...] = jnp.broadcast_to(bv2_ref[0], acc_ref.shape)

    comb = jnp.dot(xb_ref[...], wfa_ref[0], preferred_element_type=_F32)
    a = comb[:, :c_half] + bfa_ref[0, :, :c_half]
    bc = comb[:, c_half:] + bfa_ref[0, :, c_half:]
    t = ((bc * jax.nn.sigmoid(bc)) * a).astype(_BF)
    acc_ref[...] += jnp.dot(t, wb2_ref[0], preferred_element_type=_F32)

    @pl.when(j == n_chunks - 1)
    def _():
        contrib = acc_ref[...] * wts_ref[0]

        @pl.when(e == 0)
        def _():
            o_ref[...] = x_ref[...] + contrib

        @pl.when(e != 0)
        def _():
            o_ref[...] += contrib


def kernel(x, g, b, Wqkv, bqkv, alpha, Wproj, bproj, Wr, br,
           W1, b1, Wa, ba, Wb, bb, W2, b2):
    B, T, D = x.shape
    NH = alpha.shape[0]
    HD = D // NH
    E = Wr.shape[1]
    DFF = W1.shape[2]

    x2 = x.reshape(T, D)
    g2 = g.reshape(1, D)
    b2d = b.reshape(1, D)
    bqkv2 = bqkv.reshape(1, 3 * D)
    bproj2 = bproj.reshape(1, D)
    br2 = br.reshape(1, E)

    row_blk = 256 if T % 256 == 0 else T
    n_rows = T // row_blk

    qkv = pl.pallas_call(
        _qkv_body,
        grid=(n_rows,),
        in_specs=[
            pl.BlockSpec((row_blk, D), lambda i: (i, 0)),
            pl.BlockSpec((1, D), lambda i: (0, 0)),
            pl.BlockSpec((1, D), lambda i: (0, 0)),
            pl.BlockSpec((D, 3 * D), lambda i: (0, 0)),
            pl.BlockSpec((1, 3 * D), lambda i: (0, 0)),
        ],
        out_specs=pl.BlockSpec((row_blk, 3 * D), lambda i: (i, 0)),
        out_shape=jax.ShapeDtypeStruct((T, 3 * D), _F32),
    )(x2, g2, b2d, Wqkv, bqkv2)

    q_blk = row_blk

    def attn_call(m_off, n_tiles, k_rows):
        return pl.pallas_call(
            functools.partial(_attn_body, q_blk=q_blk, hd=HD, n_head=NH,
                              eps=1e-5, m_off=m_off),
            grid=(n_tiles,),
            in_specs=[
                pl.BlockSpec(memory_space=pltpu.SMEM),
                pl.BlockSpec((q_blk, D), lambda m: (m + m_off, 0)),
                pl.BlockSpec((k_rows, D), lambda m: (0, 1),
                             pipeline_mode=pl.Buffered(buffer_count=1)),
                pl.BlockSpec((k_rows, D), lambda m: (0, 2),
                             pipeline_mode=pl.Buffered(buffer_count=1)),
            ],
            out_specs=pl.BlockSpec((q_blk, D), lambda m: (m, 0)),
            out_shape=jax.ShapeDtypeStruct((n_tiles * q_blk, D), _F32),
            compiler_params=pltpu.CompilerParams(
                dimension_semantics=("arbitrary",)),
        )(alpha, qkv, qkv, qkv)

    if n_rows % 2 == 0:
        half = n_rows // 2
        y = jnp.concatenate(
            [attn_call(0, half, T // 2), attn_call(half, half, T)], axis=0)
    else:
        y = attn_call(0, n_rows, T)

    x1, x1b, w = pl.pallas_call(
        _proj_router_body,
        grid=(n_rows,),
        in_specs=[
            pl.BlockSpec((row_blk, D), lambda i: (i, 0)),
            pl.BlockSpec((D, D), lambda i: (0, 0)),
            pl.BlockSpec((1, D), lambda i: (0, 0)),
            pl.BlockSpec((row_blk, D), lambda i: (i, 0)),
            pl.BlockSpec((D, E), lambda i: (0, 0)),
            pl.BlockSpec((1, E), lambda i: (0, 0)),
        ],
        out_specs=[
            pl.BlockSpec((row_blk, D), lambda i: (i, 0)),
            pl.BlockSpec((row_blk, D), lambda i: (i, 0)),
            pl.BlockSpec((E, row_blk, 1), lambda i: (0, i, 0)),
        ],
        out_shape=[
            jax.ShapeDtypeStruct((T, D), _F32),
            jax.ShapeDtypeStruct((T, D), _BF),
            jax.ShapeDtypeStruct((E, T, 1), _F32),
        ],
    )(y, Wproj, bproj2, x2, Wr, br2)
    wts = w

    n_c = 3
    n1 = 2 * n_c
    C1 = (2 * DFF) // n1
    Wfa, bfa = pl.pallas_call(
        _wfa_body,
        grid=(E, n1),
        in_specs=[
            pl.BlockSpec((1, D, DFF), lambda e, p: (e, 0, 0)),
            pl.BlockSpec((1, DFF, C1),
                         lambda e, p: (e, 0, p // 2 + (p % 2) * n_c)),
            pl.BlockSpec((1, 1, DFF), lambda e, p: (e, 0, 0)),
            pl.BlockSpec((1, 1, C1),
                         lambda e, p: (e, 0, p // 2 + (p % 2) * n_c)),
        ],
        out_specs=[
            pl.BlockSpec((1, D, C1), lambda e, j: (e, 0, j)),
            pl.BlockSpec((1, 1, C1), lambda e, j: (e, 0, j)),
        ],
        out_shape=[
            jax.ShapeDtypeStruct((E, D, 2 * DFF), _BF),
            jax.ShapeDtypeStruct((E, 1, 2 * DFF), _F32),
        ],
        compiler_params=pltpu.CompilerParams(
            dimension_semantics=("arbitrary", "arbitrary")),
    )(W1, Wa, b1[:, None, :], ba[:, None, :])

    n2 = 4
    R2 = DFF // n2
    Wb2, bv2 = pl.pallas_call(
        _wb2_body,
        grid=(E, n2),
        in_specs=[
            pl.BlockSpec((1, R2, DFF), lambda e, r: (e, r, 0)),
            pl.BlockSpec((1, DFF, D), lambda e, r: (e, 0, 0)),
            pl.BlockSpec((1, 1, DFF), lambda e, r: (e, 0, 0)),
            pl.BlockSpec((1, 1, D), lambda e, r: (e, 0, 0)),
        ],
        out_specs=[
            pl.BlockSpec((1, R2, D), lambda e, r: (e, r, 0)),
            pl.BlockSpec((1, 1, D), lambda e, r: (e, 0, 0)),
        ],
        out_shape=[
            jax.ShapeDtypeStruct((E, DFF, D), _BF),
            jax.ShapeDtypeStruct((E, 1, D), _F32),
        ],
        compiler_params=pltpu.CompilerParams(
            dimension_semantics=("arbitrary", "arbitrary")),
    )(Wb, W2, bb[:, None, :], b2[:, None, :])

    C = DFF // n_c
    out = pl.pallas_call(
        functools.partial(_apply_body, c_half=C, n_chunks=n_c),
        grid=(E, n_c),
        in_specs=[
            pl.BlockSpec((T, D), lambda e, j: (0, 0),
                         pipeline_mode=pl.Buffered(buffer_count=1)),
            pl.BlockSpec((1, D, 2 * C), lambda e, j: (e, 0, j)),
            pl.BlockSpec((1, 1, 2 * C), lambda e, j: (e, 0, j)),
            pl.BlockSpec((1, C, D), lambda e, j: (e, j, 0)),
            pl.BlockSpec((1, 1, D), lambda e, j: (e, 0, 0)),
            pl.BlockSpec((T, D), lambda e, j: (0, 0),
                         pipeline_mode=pl.Buffered(buffer_count=1)),
            pl.BlockSpec((1, T, 1), lambda e, j: (e, 0, 0)),
        ],
        out_specs=pl.BlockSpec((T, D), lambda e, j: (0, 0)),
        out_shape=jax.ShapeDtypeStruct((T, D), _F32),
        scratch_shapes=[
            pltpu.VMEM((T, D), _F32),
        ],
        compiler_params=pltpu.CompilerParams(
            dimension_semantics=("arbitrary", "arbitrary")),
    )(x1b, Wfa, bfa, Wb2, bv2, x1, wts)

    return out.reshape(B, T, D)


# speedup vs baseline: 1.7321x; 1.0079x over previous
import functools
import jax
import jax.numpy as jnp
from jax.experimental import pallas as pl
from jax.experimental.pallas import tpu as pltpu

_BF = jnp.bfloat16
_F32 = jnp.float32


def _dot(a, b):
    return jnp.dot(a.astype(_BF), b.astype(_BF), preferred_element_type=_F32)


def _qkv_body(x_ref, g_ref, b_ref, w_ref, bias_ref, o_ref):
    x = x_ref[...]
    rms = jnp.sqrt(jnp.mean(x * x, axis=-1, keepdims=True) + 1e-6)
    xn = (x / rms) * g_ref[...] + b_ref[...]
    o_ref[...] = _dot(xn, w_ref[...]) + bias_ref[...]


def _attn_body(alpha_ref, q_ref, k_ref, v_ref, o_ref, *, q_blk, hd, n_head,
               eps, m_off):
    m = pl.program_id(0) + m_off
    rows = None
    for h in range(n_head):
        q = q_ref[:, h * hd:(h + 1) * hd]
        k = k_ref[:, h * hd:(h + 1) * hd]
        qn = jnp.sqrt(jnp.sum(q * q, axis=-1, keepdims=True))
        kn = jnp.sqrt(jnp.sum(k * k, axis=-1, keepdims=True))
        qh = (q * alpha_ref[h]) / (qn + eps)
        kh = k / (kn + eps)
        scores = jax.lax.dot_general(
            qh.astype(_BF), kh.astype(_BF),
            (((1,), (1,)), ((), ())), preferred_element_type=_F32)
        if rows is None:
            rows = m * q_blk + jax.lax.broadcasted_iota(
                jnp.int32, scores.shape, 0)
            cols = jax.lax.broadcasted_iota(jnp.int32, scores.shape, 1)
            causal = cols <= rows
        scores = jnp.where(causal, scores, jnp.float32(-1e30))
        mx = jnp.max(scores, axis=-1, keepdims=True)
        p = jnp.exp(scores - mx)
        p = p / jnp.sum(p, axis=-1, keepdims=True)
        o_ref[:, h * hd:(h + 1) * hd] = _dot(p, v_ref[:, h * hd:(h + 1) * hd])


def _proj_router_body(y_ref, wp_ref, bp_ref, x_ref, wr_ref, br_ref,
                      x1_ref, x1b_ref, w_ref):
    x1 = x_ref[...] + _dot(y_ref[...], wp_ref[...]) + bp_ref[...]
    x1_ref[...] = x1
    x1b_ref[...] = x1.astype(_BF)
    logits = jnp.dot(x1, wr_ref[...], preferred_element_type=_F32) + br_ref[...]
    mx = jnp.max(logits, axis=-1, keepdims=True)
    p = jnp.exp(logits - mx)
    p = p / jnp.sum(p, axis=-1, keepdims=True)
    p = p / jnp.sum(p, axis=-1, keepdims=True)
    w_ref[...] = jnp.transpose(p, (1, 0))[:, :, None]


def _wfa_body(w1_ref, wa_ref, b1_ref, ba_ref, wfa_ref, bfa_ref):
    wab = wa_ref[0].astype(_BF)
    wfa_ref[0] = jnp.dot(w1_ref[0].astype(_BF), wab,
                         preferred_element_type=_F32).astype(_BF)
    bfa_ref[0] = jnp.dot(b1_ref[0].astype(_BF), wab,
                         preferred_element_type=_F32) + ba_ref[0]


def _wb2_body(wb_ref, w2_ref, bb_ref, b2_ref, wb2_ref, bv2_ref):
    r = pl.program_id(1)
    w2b = w2_ref[0].astype(_BF)
    wb2_ref[0] = jnp.dot(wb_ref[0].astype(_BF), w2b,
                         preferred_element_type=_F32).astype(_BF)

    @pl.when(r == 0)
    def _():
        bv2_ref[0] = jnp.dot(bb_ref[0].astype(_BF), w2b,
                             preferred_element_type=_F32) + b2_ref[0]


def _apply_body(xb_ref, wfa_ref, bfa_ref, wb2_ref, bv2_ref, x_ref, wts_ref,
                o_ref, acc_ref, *, c_half, n_chunks):
    e = pl.program_id(0)
    j = pl.program_id(1)

    @pl.when(j == 0)
    def _():
        acc_ref[...] = jnp.broadcast_to(bv2_ref[0], acc_ref.shape)

    comb = jnp.dot(xb_ref[...], wfa_ref[0], preferred_element_type=_F32)
    a = comb[:, :c_half] + bfa_ref[0, :, :c_half]
    bc = comb[:, c_half:] + bfa_ref[0, :, c_half:]
    t = ((bc * jax.nn.sigmoid(bc)) * a).astype(_BF)
    acc_ref[...] += jnp.dot(t, wb2_ref[0], preferred_element_type=_F32)

    @pl.when(j == n_chunks - 1)
    def _():
        contrib = acc_ref[...] * wts_ref[0]

        @pl.when(e == 0)
        def _():
            o_ref[...] = x_ref[...] + contrib

        @pl.when(e != 0)
        def _():
            o_ref[...] += contrib


def kernel(x, g, b, Wqkv, bqkv, alpha, Wproj, bproj, Wr, br,
           W1, b1, Wa, ba, Wb, bb, W2, b2):
    B, T, D = x.shape
    NH = alpha.shape[0]
    HD = D // NH
    E = Wr.shape[1]
    DFF = W1.shape[2]

    x2 = x.reshape(T, D)
    g2 = g.reshape(1, D)
    b2d = b.reshape(1, D)
    bqkv2 = bqkv.reshape(1, 3 * D)
    bproj2 = bproj.reshape(1, D)
    br2 = br.reshape(1, E)

    row_blk = 256 if T % 256 == 0 else T
    n_rows = T // row_blk

    qkv = pl.pallas_call(
        _qkv_body,
        grid=(n_rows,),
        in_specs=[
            pl.BlockSpec((row_blk, D), lambda i: (i, 0)),
            pl.BlockSpec((1, D), lambda i: (0, 0)),
            pl.BlockSpec((1, D), lambda i: (0, 0)),
            pl.BlockSpec((D, 3 * D), lambda i: (0, 0)),
            pl.BlockSpec((1, 3 * D), lambda i: (0, 0)),
        ],
        out_specs=pl.BlockSpec((row_blk, 3 * D), lambda i: (i, 0)),
        out_shape=jax.ShapeDtypeStruct((T, 3 * D), _F32),
    )(x2, g2, b2d, Wqkv, bqkv2)

    q_blk = row_blk

    def attn_call(m_off, n_tiles, k_rows):
        return pl.pallas_call(
            functools.partial(_attn_body, q_blk=q_blk, hd=HD, n_head=NH,
                              eps=1e-5, m_off=m_off),
            grid=(n_tiles,),
            in_specs=[
                pl.BlockSpec(memory_space=pltpu.SMEM),
                pl.BlockSpec((q_blk, D), lambda m: (m + m_off, 0)),
                pl.BlockSpec((k_rows, D), lambda m: (0, 1),
                             pipeline_mode=pl.Buffered(buffer_count=1)),
                pl.BlockSpec((k_rows, D), lambda m: (0, 2),
                             pipeline_mode=pl.Buffered(buffer_count=1)),
            ],
            out_specs=pl.BlockSpec((q_blk, D), lambda m: (m, 0)),
            out_shape=jax.ShapeDtypeStruct((n_tiles * q_blk, D), _F32),
            compiler_params=pltpu.CompilerParams(
                dimension_semantics=("arbitrary",)),
        )(alpha, qkv, qkv, qkv)

    if n_rows % 4 == 0:
        qt = n_rows // 4
        y = jnp.concatenate(
            [attn_call(i * qt, qt, (i + 1) * (T // 4)) for i in range(4)],
            axis=0)
    elif n_rows % 2 == 0:
        half = n_rows // 2
        y = jnp.concatenate(
            [attn_call(0, half, T // 2), attn_call(half, half, T)], axis=0)
    else:
        y = attn_call(0, n_rows, T)

    x1, x1b, w = pl.pallas_call(
        _proj_router_body,
        grid=(n_rows,),
        in_specs=[
            pl.BlockSpec((row_blk, D), lambda i: (i, 0)),
            pl.BlockSpec((D, D), lambda i: (0, 0)),
            pl.BlockSpec((1, D), lambda i: (0, 0)),
            pl.BlockSpec((row_blk, D), lambda i: (i, 0)),
            pl.BlockSpec((D, E), lambda i: (0, 0)),
            pl.BlockSpec((1, E), lambda i: (0, 0)),
        ],
        out_specs=[
            pl.BlockSpec((row_blk, D), lambda i: (i, 0)),
            pl.BlockSpec((row_blk, D), lambda i: (i, 0)),
            pl.BlockSpec((E, row_blk, 1), lambda i: (0, i, 0)),
        ],
        out_shape=[
            jax.ShapeDtypeStruct((T, D), _F32),
            jax.ShapeDtypeStruct((T, D), _BF),
            jax.ShapeDtypeStruct((E, T, 1), _F32),
        ],
    )(y, Wproj, bproj2, x2, Wr, br2)
    wts = w

    n_c = 3
    n1 = 2 * n_c
    C1 = (2 * DFF) // n1
    Wfa, bfa = pl.pallas_call(
        _wfa_body,
        grid=(E, n1),
        in_specs=[
            pl.BlockSpec((1, D, DFF), lambda e, p: (e, 0, 0)),
            pl.BlockSpec((1, DFF, C1),
                         lambda e, p: (e, 0, p // 2 + (p % 2) * n_c)),
            pl.BlockSpec((1, 1, DFF), lambda e, p: (e, 0, 0)),
            pl.BlockSpec((1, 1, C1),
                         lambda e, p: (e, 0, p // 2 + (p % 2) * n_c)),
        ],
        out_specs=[
            pl.BlockSpec((1, D, C1), lambda e, j: (e, 0, j)),
            pl.BlockSpec((1, 1, C1), lambda e, j: (e, 0, j)),
        ],
        out_shape=[
            jax.ShapeDtypeStruct((E, D, 2 * DFF), _BF),
            jax.ShapeDtypeStruct((E, 1, 2 * DFF), _F32),
        ],
        compiler_params=pltpu.CompilerParams(
            dimension_semantics=("arbitrary", "arbitrary")),
    )(W1, Wa, b1[:, None, :], ba[:, None, :])

    n2 = 4
    R2 = DFF // n2
    Wb2, bv2 = pl.pallas_call(
        _wb2_body,
        grid=(E, n2),
        in_specs=[
            pl.BlockSpec((1, R2, DFF), lambda e, r: (e, r, 0)),
            pl.BlockSpec((1, DFF, D), lambda e, r: (e, 0, 0)),
            pl.BlockSpec((1, 1, DFF), lambda e, r: (e, 0, 0)),
            pl.BlockSpec((1, 1, D), lambda e, r: (e, 0, 0)),
        ],
        out_specs=[
            pl.BlockSpec((1, R2, D), lambda e, r: (e, r, 0)),
            pl.BlockSpec((1, 1, D), lambda e, r: (e, 0, 0)),
        ],
        out_shape=[
            jax.ShapeDtypeStruct((E, DFF, D), _BF),
            jax.ShapeDtypeStruct((E, 1, D), _F32),
        ],
        compiler_params=pltpu.CompilerParams(
            dimension_semantics=("arbitrary", "arbitrary")),
    )(Wb, W2, bb[:, None, :], b2[:, None, :])

    C = DFF // n_c
    out = pl.pallas_call(
        functools.partial(_apply_body, c_half=C, n_chunks=n_c),
        grid=(E, n_c),
        in_specs=[
            pl.BlockSpec((T, D), lambda e, j: (0, 0),
                         pipeline_mode=pl.Buffered(buffer_count=1)),
            pl.BlockSpec((1, D, 2 * C), lambda e, j: (e, 0, j)),
            pl.BlockSpec((1, 1, 2 * C), lambda e, j: (e, 0, j)),
            pl.BlockSpec((1, C, D), lambda e, j: (e, j, 0)),
            pl.BlockSpec((1, 1, D), lambda e, j: (e, 0, 0)),
            pl.BlockSpec((T, D), lambda e, j: (0, 0),
                         pipeline_mode=pl.Buffered(buffer_count=1)),
            pl.BlockSpec((1, T, 1), lambda e, j: (e, 0, 0)),
        ],
        out_specs=pl.BlockSpec((T, D), lambda e, j: (0, 0)),
        out_shape=jax.ShapeDtypeStruct((T, D), _F32),
        scratch_shapes=[
            pltpu.VMEM((T, D), _F32),
        ],
        compiler_params=pltpu.CompilerParams(
            dimension_semantics=("arbitrary", "arbitrary")),
    )(x1b, Wfa, bfa, Wb2, bv2, x1, wts)

    return out.reshape(B, T, D)


# speedup vs baseline: 1.7339x; 1.0010x over previous
import functools
import jax
import jax.numpy as jnp
from jax.experimental import pallas as pl
from jax.experimental.pallas import tpu as pltpu

_BF = jnp.bfloat16
_F32 = jnp.float32


def _dot(a, b):
    return jnp.dot(a.astype(_BF), b.astype(_BF), preferred_element_type=_F32)


def _qkv_body(x_ref, g_ref, b_ref, w_ref, bias_ref, o_ref):
    x = x_ref[...]
    rms = jnp.sqrt(jnp.mean(x * x, axis=-1, keepdims=True) + 1e-6)
    xn = (x / rms) * g_ref[...] + b_ref[...]
    o_ref[...] = _dot(xn, w_ref[...]) + bias_ref[...]


def _attn_body(alpha_ref, q_ref, k_ref, v_ref, o_ref, *, q_blk, hd, n_head,
               eps, m_off):
    m = pl.program_id(0) + m_off
    rows = None
    for h in range(n_head):
        q = q_ref[:, h * hd:(h + 1) * hd]
        k = k_ref[:, h * hd:(h + 1) * hd]
        qn = jnp.sqrt(jnp.sum(q * q, axis=-1, keepdims=True))
        kn = jnp.sqrt(jnp.sum(k * k, axis=-1, keepdims=True))
        qh = (q * alpha_ref[h]) / (qn + eps)
        kh = k / (kn + eps)
        scores = jax.lax.dot_general(
            qh.astype(_BF), kh.astype(_BF),
            (((1,), (1,)), ((), ())), preferred_element_type=_F32)
        if rows is None:
            rows = m * q_blk + jax.lax.broadcasted_iota(
                jnp.int32, scores.shape, 0)
            cols = jax.lax.broadcasted_iota(jnp.int32, scores.shape, 1)
            causal = cols <= rows
        scores = jnp.where(causal, scores, jnp.float32(-1e30))
        mx = jnp.max(scores, axis=-1, keepdims=True)
        p = jnp.exp(scores - mx)
        p = p / jnp.sum(p, axis=-1, keepdims=True)
        o_ref[:, h * hd:(h + 1) * hd] = _dot(p, v_ref[:, h * hd:(h + 1) * hd])


def _proj_router_body(y_ref, wp_ref, bp_ref, x_ref, wr_ref, br_ref,
                      x1_ref, x1b_ref, w_ref):
    x1 = x_ref[...] + _dot(y_ref[...], wp_ref[...]) + bp_ref[...]
    x1_ref[...] = x1
    x1b_ref[...] = x1.astype(_BF)
    logits = jnp.dot(x1, wr_ref[...], preferred_element_type=_F32) + br_ref[...]
    mx = jnp.max(logits, axis=-1, keepdims=True)
    p = jnp.exp(logits - mx)
    p = p / jnp.sum(p, axis=-1, keepdims=True)
    p = p / jnp.sum(p, axis=-1, keepdims=True)
    w_ref[...] = jnp.transpose(p, (1, 0))[:, :, None]


def _wfa_body(w1_ref, wat_ref, wab_ref, b1_ref, ba_ref, wfa_ref, bfa_ref,
              *, kh):
    w1b = w1_ref[0].astype(_BF)
    b1b = b1_ref[0].astype(_BF)
    top = wat_ref[0].astype(_BF)
    bot = wab_ref[0].astype(_BF)
    wfa_ref[0] = (
        jnp.dot(w1b[:, :kh], top, preferred_element_type=_F32)
        + jnp.dot(w1b[:, kh:], bot, preferred_element_type=_F32)
    ).astype(_BF)
    bfa_ref[0] = (jnp.dot(b1b[:, :kh], top, preferred_element_type=_F32)
                  + jnp.dot(b1b[:, kh:], bot, preferred_element_type=_F32)
                  + ba_ref[0])


def _wb2_body(wbl_ref, wbr_ref, w2_ref, bb_ref, b2_ref, wb2_ref, bv2_ref,
              *, kh):
    r = pl.program_id(1)
    w2b = w2_ref[0].astype(_BF)
    wb2_ref[0] = (
        jnp.dot(wbl_ref[0].astype(_BF), w2b[:kh],
                preferred_element_type=_F32)
        + jnp.dot(wbr_ref[0].astype(_BF), w2b[kh:],
                  preferred_element_type=_F32)
    ).astype(_BF)

    @pl.when(r == 0)
    def _():
        bv2_ref[0] = jnp.dot(bb_ref[0].astype(_BF), w2b,
                             preferred_element_type=_F32) + b2_ref[0]


def _apply_body(xb_ref, wfa_ref, bfa_ref, wb2_ref, bv2_ref, x_ref, wts_ref,
                o_ref, acc_ref, *, c_half, n_chunks):
    e = pl.program_id(0)
    j = pl.program_id(1)

    @pl.when(j == 0)
    def _():
        acc_ref[...] = jnp.broadcast_to(bv2_ref[0], acc_ref.shape)

    comb = jnp.dot(xb_ref[...], wfa_ref[0], preferred_element_type=_F32)
    a = comb[:, :c_half] + bfa_ref[0, :, :c_half]
    bc = comb[:, c_half:] + bfa_ref[0, :, c_half:]
    t = ((bc * jax.nn.sigmoid(bc)) * a).astype(_BF)
    acc_ref[...] += jnp.dot(t, wb2_ref[0], preferred_element_type=_F32)

    @pl.when(j == n_chunks - 1)
    def _():
        contrib = acc_ref[...] * wts_ref[0]

        @pl.when(e == 0)
        def _():
            o_ref[...] = x_ref[...] + contrib

        @pl.when(e != 0)
        def _():
            o_ref[...] += contrib


def kernel(x, g, b, Wqkv, bqkv, alpha, Wproj, bproj, Wr, br,
           W1, b1, Wa, ba, Wb, bb, W2, b2):
    B, T, D = x.shape
    NH = alpha.shape[0]
    HD = D // NH
    E = Wr.shape[1]
    DFF = W1.shape[2]

    x2 = x.reshape(T, D)
    g2 = g.reshape(1, D)
    b2d = b.reshape(1, D)
    bqkv2 = bqkv.reshape(1, 3 * D)
    bproj2 = bproj.reshape(1, D)
    br2 = br.reshape(1, E)

    row_blk = 256 if T % 256 == 0 else T
    n_rows = T // row_blk

    qkv = pl.pallas_call(
        _qkv_body,
        grid=(n_rows,),
        in_specs=[
            pl.BlockSpec((row_blk, D), lambda i: (i, 0)),
            pl.BlockSpec((1, D), lambda i: (0, 0)),
            pl.BlockSpec((1, D), lambda i: (0, 0)),
            pl.BlockSpec((D, 3 * D), lambda i: (0, 0)),
            pl.BlockSpec((1, 3 * D), lambda i: (0, 0)),
        ],
        out_specs=pl.BlockSpec((row_blk, 3 * D), lambda i: (i, 0)),
        out_shape=jax.ShapeDtypeStruct((T, 3 * D), _F32),
    )(x2, g2, b2d, Wqkv, bqkv2)

    q_blk = row_blk

    def attn_call(m_off, n_tiles, k_rows):
        return pl.pallas_call(
            functools.partial(_attn_body, q_blk=q_blk, hd=HD, n_head=NH,
                              eps=1e-5, m_off=m_off),
            grid=(n_tiles,),
            in_specs=[
                pl.BlockSpec(memory_space=pltpu.SMEM),
                pl.BlockSpec((q_blk, D), lambda m: (m + m_off, 0)),
                pl.BlockSpec((k_rows, D), lambda m: (0, 1),
                             pipeline_mode=pl.Buffered(buffer_count=1)),
                pl.BlockSpec((k_rows, D), lambda m: (0, 2),
                             pipeline_mode=pl.Buffered(buffer_count=1)),
            ],
            out_specs=pl.BlockSpec((q_blk, D), lambda m: (m, 0)),
            out_shape=jax.ShapeDtypeStruct((n_tiles * q_blk, D), _F32),
            compiler_params=pltpu.CompilerParams(
                dimension_semantics=("arbitrary",)),
        )(alpha, qkv, qkv, qkv)

    if n_rows % 4 == 0:
        qt = n_rows // 4
        y = jnp.concatenate(
            [attn_call(i * qt, qt, (i + 1) * (T // 4)) for i in range(4)],
            axis=0)
    elif n_rows % 2 == 0:
        half = n_rows // 2
        y = jnp.concatenate(
            [attn_call(0, half, T // 2), attn_call(half, half, T)], axis=0)
    else:
        y = attn_call(0, n_rows, T)

    x1, x1b, w = pl.pallas_call(
        _proj_router_body,
        grid=(n_rows,),
        in_specs=[
            pl.BlockSpec((row_blk, D), lambda i: (i, 0)),
            pl.BlockSpec((D, D), lambda i: (0, 0)),
            pl.BlockSpec((1, D), lambda i: (0, 0)),
            pl.BlockSpec((row_blk, D), lambda i: (i, 0)),
            pl.BlockSpec((D, E), lambda i: (0, 0)),
            pl.BlockSpec((1, E), lambda i: (0, 0)),
        ],
        out_specs=[
            pl.BlockSpec((row_blk, D), lambda i: (i, 0)),
            pl.BlockSpec((row_blk, D), lambda i: (i, 0)),
            pl.BlockSpec((E, row_blk, 1), lambda i: (0, i, 0)),
        ],
        out_shape=[
            jax.ShapeDtypeStruct((T, D), _F32),
            jax.ShapeDtypeStruct((T, D), _BF),
            jax.ShapeDtypeStruct((E, T, 1), _F32),
        ],
    )(y, Wproj, bproj2, x2, Wr, br2)
    wts = w

    n_c = 3
    n1 = 2 * n_c
    C1 = (2 * DFF) // n1
    Wfa, bfa = pl.pallas_call(
        functools.partial(_wfa_body, kh=DFF // 2),
        grid=(E, n1),
        in_specs=[
            pl.BlockSpec((1, D, DFF), lambda e, p: (e, 0, 0)),
            pl.BlockSpec((1, DFF // 2, C1),
                         lambda e, p: (e, 0, p // 2 + (p % 2) * n_c)),
            pl.BlockSpec((1, DFF // 2, C1),
                         lambda e, p: (e, 1, p // 2 + (p % 2) * n_c)),
            pl.BlockSpec((1, 1, DFF), lambda e, p: (e, 0, 0)),
            pl.BlockSpec((1, 1, C1),
                         lambda e, p: (e, 0, p // 2 + (p % 2) * n_c)),
        ],
        out_specs=[
            pl.BlockSpec((1, D, C1), lambda e, j: (e, 0, j)),
            pl.BlockSpec((1, 1, C1), lambda e, j: (e, 0, j)),
        ],
        out_shape=[
            jax.ShapeDtypeStruct((E, D, 2 * DFF), _BF),
            jax.ShapeDtypeStruct((E, 1, 2 * DFF), _F32),
        ],
        compiler_params=pltpu.CompilerParams(
            dimension_semantics=("arbitrary", "arbitrary")),
    )(W1, Wa, Wa, b1[:, None, :], ba[:, None, :])

    n2 = 4
    R2 = DFF // n2
    Wb2, bv2 = pl.pallas_call(
        functools.partial(_wb2_body, kh=DFF // 2),
        grid=(E, n2),
        in_specs=[
            pl.BlockSpec((1, R2, DFF // 2), lambda e, r: (e, r, 0)),
            pl.BlockSpec((1, R2, DFF // 2), lambda e, r: (e, r, 1)),
            pl.BlockSpec((1, DFF, D), lambda e, r: (e, 0, 0)),
            pl.BlockSpec((1, 1, DFF), lambda e, r: (e, 0, 0)),
            pl.BlockSpec((1, 1, D), lambda e, r: (e, 0, 0)),
        ],
        out_specs=[
            pl.BlockSpec((1, R2, D), lambda e, r: (e, r, 0)),
            pl.BlockSpec((1, 1, D), lambda e, r: (e, 0, 0)),
        ],
        out_shape=[
            jax.ShapeDtypeStruct((E, DFF, D), _BF),
            jax.ShapeDtypeStruct((E, 1, D), _F32),
        ],
        compiler_params=pltpu.CompilerParams(
            dimension_semantics=("arbitrary", "arbitrary")),
    )(Wb, Wb, W2, bb[:, None, :], b2[:, None, :])

    C = DFF // n_c
    out = pl.pallas_call(
        functools.partial(_apply_body, c_half=C, n_chunks=n_c),
        grid=(E, n_c),
        in_specs=[
            pl.BlockSpec((T, D), lambda e, j: (0, 0),
                         pipeline_mode=pl.Buffered(buffer_count=1)),
            pl.BlockSpec((1, D, 2 * C), lambda e, j: (e, 0, j)),
            pl.BlockSpec((1, 1, 2 * C), lambda e, j: (e, 0, j)),
            pl.BlockSpec((1, C, D), lambda e, j: (e, j, 0)),
            pl.BlockSpec((1, 1, D), lambda e, j: (e, 0, 0)),
            pl.BlockSpec((T, D), lambda e, j: (0, 0),
                         pipeline_mode=pl.Buffered(buffer_count=1)),
            pl.BlockSpec((1, T, 1), lambda e, j: (e, 0, 0)),
        ],
        out_specs=pl.BlockSpec((T, D), lambda e, j: (0, 0)),
        out_shape=jax.ShapeDtypeStruct((T, D), _F32),
        scratch_shapes=[
            pltpu.VMEM((T, D), _F32),
        ],
        compiler_params=pltpu.CompilerParams(
            dimension_semantics=("arbitrary", "arbitrary")),
    )(x1b, Wfa, bfa, Wb2, bv2, x1, wts)

    return out.reshape(B, T, D)
